# Initial kernel scaffold; baseline (speedup 1.0000x reference)
import jax
import jax.numpy as jnp
from jax.experimental import pallas as pl


def kernel(feat_index, feat_value, first_weights, feat_embeddings, bias):
    raise NotImplementedError("write your pallas kernel here")


# speedup vs baseline: 1.3250x; 1.3250x over previous
import functools

import jax
import jax.numpy as jnp
from jax import lax
from jax.experimental import pallas as pl
from jax.experimental.pallas import tpu as pltpu
from jax.experimental.pallas import tpu_sc as plsc

B = 16384
F = 26
E = 16
L = 16
NW = 32
RPW = B // NW
CHUNK = 64
NCH = RPW // CHUNK
IDXR = CHUNK * F // 128
NIDX = CHUNK * F
GPC = CHUNK // L


def _lane_shuffle(x, p):
    dnums = lax.GatherDimensionNumbers(
        offset_dims=(), collapsed_slice_dims=(0,), start_index_map=(0,))
    return lax.gather(x, p[:, None], dnums, slice_sizes=(1,),
                      mode=lax.GatherScatterMode.PROMISE_IN_BOUNDS)


def _fm_body(idx_hbm, fv_hbm, fw_hbm, emb_hbm, bias_hbm, out_hbm,
             idx_v, fv_v, fw_v, emb_v, bias_v, out_v, sem):
    cid = lax.axis_index("c")
    sid = lax.axis_index("s")
    wid = sid * 2 + cid
    row0 = wid * RPW
    iota = lax.iota(jnp.int32, L)
    masks = [iota == r for r in range(L)]
    tailm = iota >= (2 * L - F)
    perms = [iota ^ k for k in (8, 4, 2, 1)]

    pltpu.sync_copy(bias_hbm, bias_v)
    bvec = bias_v[...]
    pltpu.sync_copy(idx_hbm.at[pl.ds(wid * (NCH * IDXR), NCH * IDXR)], idx_v)

    def chunk_body(c, carry):
        pltpu.sync_copy(fv_hbm.at[pl.ds((row0 + c * CHUNK) * F, NIDX)], fv_v)

        descs = []
        for j in range(IDXR):
            descs.append(pltpu.async_copy(
                emb_hbm.at[idx_v.at[c * IDXR + j]],
                emb_v.at[pl.ds(j * 128, 128)], sem))
            descs.append(pltpu.async_copy(
                fw_hbm.at[idx_v.at[c * IDXR + j]],
                fw_v.at[pl.ds(j * 128, 128)], sem))
        for d in descs:
            d.wait()

        def group_body(g, gcarry):
            acc = bvec
            for r in range(L):
                base = (g * L + r) * F
                v1 = fv_v[pl.ds(base, L)]
                v2 = fv_v[pl.ds(base + F - L, L)]
                w1 = fw_v[pl.ds(base, L)]
                w2 = fw_v[pl.ds(base + F - L, L)]
                s = jnp.zeros((L,), jnp.float32)
                q = jnp.zeros((L,), jnp.float32)
                for f in range(F):
                    v = v1[f] if f < L else v2[f - (F - L)]
                    row = emb_v.at[base + f][...]
                    t = row * v
                    s = s + t
                    q = q + t * t
                fo = w1 * v1 + jnp.where(tailm, w2 * v2, jnp.float32(0.0))
                u = fo + 0.5 * (s * s - q)
                for p in perms:
                    u = u + _lane_shuffle(u, p)
                acc = acc + jnp.where(masks[r], u, jnp.float32(0.0))
            out_v[pl.ds(c * CHUNK + g * L, L)] = acc
            return gcarry

        lax.fori_loop(0, GPC, group_body, 0)
        return carry

    lax.fori_loop(0, NCH, chunk_body, 0)
    pltpu.sync_copy(out_v, out_hbm.at[pl.ds(row0, RPW)])


_fm_kernel = pl.kernel(
    _fm_body,
    out_type=jax.ShapeDtypeStruct((B,), jnp.float32),
    mesh=plsc.VectorSubcoreMesh(core_axis_name="c", subcore_axis_name="s"),
    compiler_params=pltpu.CompilerParams(use_tc_tiling_on_sc=False),
    scratch_types=[
        pltpu.VMEM((NCH * IDXR, 128), jnp.int32),
        pltpu.VMEM((NIDX,), jnp.float32),
        pltpu.VMEM((NIDX,), jnp.float32),
        pltpu.VMEM((NIDX, E), jnp.float32),
        pltpu.VMEM((L,), jnp.float32),
        pltpu.VMEM((RPW,), jnp.float32),
        pltpu.SemaphoreType.DMA,
    ],
)


@jax.jit
def kernel(feat_index, feat_value, first_weights, feat_embeddings, bias):
    idx2d = feat_index.reshape(B * F // 128, 128)
    fv_flat = feat_value.reshape(B * F)
    fw_flat = first_weights.reshape(first_weights.shape[0])
    bias16 = jnp.broadcast_to(bias, (L,))
    out = _fm_kernel(idx2d, fv_flat, fw_flat, feat_embeddings,
                     bias16)
    return out[:, None]


# speedup vs baseline: 1.3264x; 1.0011x over previous
import functools

import jax
import jax.numpy as jnp
from jax import lax
from jax.experimental import pallas as pl
from jax.experimental.pallas import tpu as pltpu
from jax.experimental.pallas import tpu_sc as plsc

B = 16384
F = 26
E = 16
L = 16
NW = 32
RPW = B // NW
CHUNK = 64
NCH = RPW // CHUNK
NIDX = CHUNK * F
GPC = CHUNK // L


def _lane_shuffle(x, p):
    dnums = lax.GatherDimensionNumbers(
        offset_dims=(), collapsed_slice_dims=(0,), start_index_map=(0,))
    return lax.gather(x, p[:, None], dnums, slice_sizes=(1,),
                      mode=lax.GatherScatterMode.PROMISE_IN_BOUNDS)


def _fm_body(idx_hbm, fv_hbm, fw_hbm, emb_hbm, bias_hbm, out_hbm,
             idx_v, fv_v, fw_v, emb_v, bias_v, out_v, sem):
    cid = lax.axis_index("c")
    sid = lax.axis_index("s")
    wid = sid * 2 + cid
    row0 = wid * RPW
    iota = lax.iota(jnp.int32, L)
    masks = [iota == r for r in range(L)]
    tailm = iota >= (2 * L - F)
    perms = [iota ^ k for k in (8, 4, 2, 1)]

    pltpu.sync_copy(bias_hbm, bias_v)
    bvec = bias_v[...]
    pltpu.sync_copy(idx_hbm.at[pl.ds(wid * (NCH * NIDX), NCH * NIDX)], idx_v)

    def chunk_body(c, carry):
        pltpu.sync_copy(fv_hbm.at[pl.ds((row0 + c * CHUNK) * F, NIDX)], fv_v)

        idx_c = idx_v.at[pl.ds(c * NIDX, NIDX)]
        d1 = pltpu.async_copy(emb_hbm.at[idx_c], emb_v, sem)
        d2 = pltpu.async_copy(fw_hbm.at[idx_c], fw_v, sem)
        d1.wait()
        d2.wait()

        def group_body(g, gcarry):
            acc = bvec
            for r in range(L):
                base = (g * L + r) * F
                v1 = fv_v[pl.ds(base, L)]
                v2 = fv_v[pl.ds(base + F - L, L)]
                w1 = fw_v[pl.ds(base, L)]
                w2 = fw_v[pl.ds(base + F - L, L)]
                s = jnp.zeros((L,), jnp.float32)
                q = jnp.zeros((L,), jnp.float32)
                for f in range(F):
                    v = v1[f] if f < L else v2[f - (F - L)]
                    row = emb_v.at[base + f][...]
                    t = row * v
                    s = s + t
                    q = q + t * t
                fo = w1 * v1 + jnp.where(tailm, w2 * v2, jnp.float32(0.0))
                u = fo + 0.5 * (s * s - q)
                for p in perms:
                    u = u + _lane_shuffle(u, p)
                acc = acc + jnp.where(masks[r], u, jnp.float32(0.0))
            out_v[pl.ds(c * CHUNK + g * L, L)] = acc
            return gcarry

        lax.fori_loop(0, GPC, group_body, 0)
        return carry

    lax.fori_loop(0, NCH, chunk_body, 0)
    pltpu.sync_copy(out_v, out_hbm.at[pl.ds(row0, RPW)])


_fm_kernel = pl.kernel(
    _fm_body,
    out_type=jax.ShapeDtypeStruct((B,), jnp.float32),
    mesh=plsc.VectorSubcoreMesh(core_axis_name="c", subcore_axis_name="s"),
    compiler_params=pltpu.CompilerParams(use_tc_tiling_on_sc=False),
    scratch_types=[
        pltpu.VMEM((NCH * NIDX,), jnp.int32),
        pltpu.VMEM((NIDX,), jnp.float32),
        pltpu.VMEM((NIDX,), jnp.float32),
        pltpu.VMEM((NIDX, E), jnp.float32),
        pltpu.VMEM((L,), jnp.float32),
        pltpu.VMEM((RPW,), jnp.float32),
        pltpu.SemaphoreType.DMA,
    ],
)


@jax.jit
def kernel(feat_index, feat_value, first_weights, feat_embeddings, bias):
    idx_flat = feat_index.reshape(B * F)
    fv_flat = feat_value.reshape(B * F)
    fw_flat = first_weights.reshape(first_weights.shape[0])
    bias16 = jnp.broadcast_to(bias, (L,))
    out = _fm_kernel(idx_flat, fv_flat, fw_flat, feat_embeddings,
                     bias16)
    return out[:, None]
